# baseline (speedup 1.0000x reference)
import functools

import jax
import jax.numpy as jnp
from jax import lax
from jax.experimental import pallas as pl
from jax.experimental.pallas import tpu as pltpu
from jax.experimental.pallas import tpu_sc as plsc

NUM_EXPERTS = 8
TOP_K = 2
HIDDEN = 768
INTER = 2048
T_TOKENS = 2048
N_PAIRS = T_TOKENS * TOP_K

TB = 256
BM = 128
G = N_PAIRS // BM + NUM_EXPERTS
R = G * BM

SC_CORES = 2
SC_SUBCORES = 16
NW = SC_CORES * SC_SUBCORES
LANES = 16


def _router_body(x_ref, gw_ref, idx_ref, w_ref):
    x = x_ref[...]
    logits = jnp.dot(x, gw_ref[...], preferred_element_type=jnp.float32)
    m = jnp.max(logits, axis=1, keepdims=True)
    ex = jnp.exp(logits - m)
    p = ex / jnp.sum(ex, axis=1, keepdims=True)
    iota = jax.lax.broadcasted_iota(jnp.int32, (p.shape[0], NUM_EXPERTS), 1)
    m1 = jnp.max(p, axis=1, keepdims=True)
    i1 = jnp.min(jnp.where(p == m1, iota, NUM_EXPERTS), axis=1, keepdims=True)
    is1 = iota == i1
    p2 = jnp.where(is1, -1.0, p)
    m2 = jnp.max(p2, axis=1, keepdims=True)
    i2 = jnp.min(jnp.where(p2 == m2, iota, NUM_EXPERTS), axis=1, keepdims=True)
    s = m1 + m2
    idx_ref[...] = jnp.concatenate([i1, i2], axis=1)
    w_ref[...] = jnp.concatenate([m1 / s, m2 / s], axis=1)


def _grouped_body(be_ref, xs_ref, w1_ref, w3_ref, w2_ref, rw_ref, ys_ref):
    g = pl.program_id(0)

    @pl.when(be_ref[g] < NUM_EXPERTS)
    def _():
        xb = xs_ref[...]
        h1 = jnp.dot(xb, w1_ref[0], preferred_element_type=jnp.float32)
        h3 = jnp.dot(xb, w3_ref[0], preferred_element_type=jnp.float32)
        hh = h1 * jax.nn.sigmoid(h1) * h3
        y = jnp.dot(hh, w2_ref[0], preferred_element_type=jnp.float32)
        ys_ref[...] = y * rw_ref[...]


GCH = 4
GCR = R // NW // GCH


def _sc_gather_body(table_hbm, idx_hbm, out_hbm, idx_v, rows0, rows1,
                    rows2, rows3, gsem0, gsem1, gsem2, gsem3, wsem):
    wid = lax.axis_index("s") * SC_CORES + lax.axis_index("c")
    bw = R // NW
    base = wid * bw
    pltpu.sync_copy(idx_hbm.at[pl.ds(wid * GCH, GCH)], idx_v)
    bufs = (rows0, rows1, rows2, rows3)
    gsems = (gsem0, gsem1, gsem2, gsem3)
    gath = [pltpu.async_copy(table_hbm.at[idx_v.at[c]], bufs[c], gsems[c])
            for c in range(GCH)]
    writes = []
    for c in range(GCH):
        gath[c].wait()
        writes.append(pltpu.async_copy(
            bufs[c], out_hbm.at[pl.ds(base + c * GCR, GCR)], wsem))
    for h in writes:
        h.wait()


def _sc_combine_body(ys_hbm, pos_hbm, out_hbm, idx_v, rows_v, out_v, sem):
    wid = lax.axis_index("s") * SC_CORES + lax.axis_index("c")
    tpw = T_TOKENS // NW
    chunk = 32

    for c in range(tpw // chunk):
        tok0 = wid * tpw + c * chunk
        pltpu.sync_copy(pos_hbm.at[pl.ds(tok0 * TOP_K, chunk * TOP_K)], idx_v)
        pltpu.async_copy(ys_hbm.at[idx_v], rows_v, sem).wait()

        def tok_body(i, carry):
            for j in range(HIDDEN // LANES):
                a = rows_v[2 * i, pl.ds(j * LANES, LANES)]
                b = rows_v[2 * i + 1, pl.ds(j * LANES, LANES)]
                out_v[i, pl.ds(j * LANES, LANES)] = a + b
            return carry

        lax.fori_loop(0, chunk, tok_body, 0)
        pltpu.sync_copy(out_v, out_hbm.at[pl.ds(tok0, chunk)])


def kernel(hidden_states, gate_w, w1s, w2s, w3s):
    B, S, H = hidden_states.shape
    x = hidden_states.reshape(-1, H)

    topi, topw = pl.pallas_call(
        _router_body,
        grid=(T_TOKENS // TB,),
        in_specs=[
            pl.BlockSpec((TB, HIDDEN), lambda t: (t, 0)),
            pl.BlockSpec((HIDDEN, NUM_EXPERTS), lambda t: (0, 0)),
        ],
        out_specs=[
            pl.BlockSpec((TB, TOP_K), lambda t: (t, 0)),
            pl.BlockSpec((TB, TOP_K), lambda t: (t, 0)),
        ],
        out_shape=[
            jax.ShapeDtypeStruct((T_TOKENS, TOP_K), jnp.int32),
            jax.ShapeDtypeStruct((T_TOKENS, TOP_K), jnp.float32),
        ],
    )(x, gate_w)

    ef = topi.reshape(-1)
    wf = topw.reshape(-1)
    onehot = (ef[:, None] == jnp.arange(NUM_EXPERTS)[None, :]).astype(jnp.int32)
    csum = jnp.cumsum(onehot, axis=0)
    counts = csum[-1]
    pos_in_e = jnp.sum(onehot * csum, axis=1) - 1
    nb = (counts + BM - 1) // BM
    bstart = jnp.concatenate([jnp.zeros((1,), nb.dtype), jnp.cumsum(nb)[:-1]])
    pad_start = (bstart * BM).astype(jnp.int32)
    pos = jnp.sum(onehot * pad_start[None, :], axis=1) + pos_in_e
    tok_of_pair = (jnp.arange(N_PAIRS, dtype=jnp.int32) // TOP_K)
    row_token = jnp.zeros((R,), jnp.int32).at[pos].set(
        tok_of_pair, unique_indices=True)
    row_w = jnp.zeros((R,), jnp.float32).at[pos].set(wf, unique_indices=True)
    nb_total = bstart[-1] + nb[-1]
    gids = jnp.arange(G, dtype=jnp.int32)
    be = (jnp.sum(gids[:, None] >= bstart[None, :], axis=1) - 1).astype(jnp.int32)
    block_expert = jnp.where(gids < nb_total, be, NUM_EXPERTS)

    sc_gather = pl.kernel(
        _sc_gather_body,
        out_type=jax.ShapeDtypeStruct((R, HIDDEN), jnp.float32),
        mesh=plsc.VectorSubcoreMesh(core_axis_name="c", subcore_axis_name="s",
                               num_cores=SC_CORES, num_subcores=SC_SUBCORES),
        scratch_types=[
            pltpu.VMEM((GCH, GCR), jnp.int32),
            pltpu.VMEM((GCR, HIDDEN), jnp.float32),
            pltpu.VMEM((GCR, HIDDEN), jnp.float32),
            pltpu.VMEM((GCR, HIDDEN), jnp.float32),
            pltpu.VMEM((GCR, HIDDEN), jnp.float32),
            pltpu.SemaphoreType.DMA,
            pltpu.SemaphoreType.DMA,
            pltpu.SemaphoreType.DMA,
            pltpu.SemaphoreType.DMA,
            pltpu.SemaphoreType.DMA,
        ],
    )
    xs = sc_gather(x, row_token.reshape(R // GCR, GCR))

    grid_spec = pltpu.PrefetchScalarGridSpec(
        num_scalar_prefetch=1,
        grid=(G,),
        in_specs=[
            pl.BlockSpec((BM, HIDDEN), lambda g, be: (g, 0)),
            pl.BlockSpec((1, HIDDEN, INTER),
                         lambda g, be: (jnp.minimum(be[g], NUM_EXPERTS - 1), 0, 0)),
            pl.BlockSpec((1, HIDDEN, INTER),
                         lambda g, be: (jnp.minimum(be[g], NUM_EXPERTS - 1), 0, 0)),
            pl.BlockSpec((1, INTER, HIDDEN),
                         lambda g, be: (jnp.minimum(be[g], NUM_EXPERTS - 1), 0, 0)),
            pl.BlockSpec((BM, 1), lambda g, be: (g, 0)),
        ],
        out_specs=pl.BlockSpec((BM, HIDDEN), lambda g, be: (g, 0)),
    )
    ys = pl.pallas_call(
        _grouped_body,
        grid_spec=grid_spec,
        out_shape=jax.ShapeDtypeStruct((R, HIDDEN), jnp.float32),
        compiler_params=pltpu.CompilerParams(
            dimension_semantics=("arbitrary",),
        ),
    )(block_expert, xs, w1s, w3s, w2s, row_w.reshape(R, 1))

    sc_combine = pl.kernel(
        _sc_combine_body,
        out_type=jax.ShapeDtypeStruct((T_TOKENS, HIDDEN), jnp.float32),
        mesh=plsc.VectorSubcoreMesh(core_axis_name="c", subcore_axis_name="s",
                               num_cores=SC_CORES, num_subcores=SC_SUBCORES),
        scratch_types=[
            pltpu.VMEM((64,), jnp.int32),
            pltpu.VMEM((64, HIDDEN), jnp.float32),
            pltpu.VMEM((32, HIDDEN), jnp.float32),
            pltpu.SemaphoreType.DMA,
        ],
    )
    out = sc_combine(ys, pos.astype(jnp.int32))

    return out.reshape(B, S, H)


# speedup vs baseline: 1.3187x; 1.3187x over previous
import functools

import jax
import jax.numpy as jnp
from jax import lax
from jax.experimental import pallas as pl
from jax.experimental.pallas import tpu as pltpu
from jax.experimental.pallas import tpu_sc as plsc

NUM_EXPERTS = 8
TOP_K = 2
HIDDEN = 768
INTER = 2048
T_TOKENS = 2048
N_PAIRS = T_TOKENS * TOP_K

TB = 256
BM = 128
G = N_PAIRS // BM + NUM_EXPERTS
R = G * BM

SC_CORES = 2
SC_SUBCORES = 16
NW = SC_CORES * SC_SUBCORES
LANES = 16


def _router_body(x_ref, gw_ref, idx_ref, w_ref):
    x = x_ref[...]
    logits = jnp.dot(x, gw_ref[...], preferred_element_type=jnp.float32)
    m = jnp.max(logits, axis=1, keepdims=True)
    ex = jnp.exp(logits - m)
    p = ex / jnp.sum(ex, axis=1, keepdims=True)
    iota = jax.lax.broadcasted_iota(jnp.int32, (p.shape[0], NUM_EXPERTS), 1)
    m1 = jnp.max(p, axis=1, keepdims=True)
    i1 = jnp.min(jnp.where(p == m1, iota, NUM_EXPERTS), axis=1, keepdims=True)
    is1 = iota == i1
    p2 = jnp.where(is1, -1.0, p)
    m2 = jnp.max(p2, axis=1, keepdims=True)
    i2 = jnp.min(jnp.where(p2 == m2, iota, NUM_EXPERTS), axis=1, keepdims=True)
    s = m1 + m2
    idx_ref[...] = jnp.concatenate([i1, i2], axis=1)
    w_ref[...] = jnp.concatenate([m1 / s, m2 / s], axis=1)


def _grouped_body(be_ref, xs_ref, w1_ref, w3_ref, w2_ref, rw_ref, ys_ref):
    g = pl.program_id(0)

    @pl.when(be_ref[g] < NUM_EXPERTS)
    def _():
        xb = xs_ref[...]
        h1 = jnp.dot(xb, w1_ref[0], preferred_element_type=jnp.float32)
        h3 = jnp.dot(xb, w3_ref[0], preferred_element_type=jnp.float32)
        hh = h1 * jax.nn.sigmoid(h1) * h3
        y = jnp.dot(hh, w2_ref[0], preferred_element_type=jnp.float32)
        ys_ref[...] = y * rw_ref[...]


TPW = T_TOKENS // NW


def _sc_distribute_body(x_hbm, pos_hbm, xs_hbm, xrows_v, idx_v, sem0, sem1):
    wid = lax.axis_index("s") * SC_CORES + lax.axis_index("c")
    pltpu.sync_copy(x_hbm.at[pl.ds(wid * TPW, TPW)], xrows_v)
    pltpu.sync_copy(pos_hbm.at[wid], idx_v)
    c0 = pltpu.async_copy(xrows_v, xs_hbm.at[idx_v.at[0]], sem0)
    c1 = pltpu.async_copy(xrows_v, xs_hbm.at[idx_v.at[1]], sem1)
    c0.wait()
    c1.wait()


def _sc_combine_body(ys_hbm, pos_hbm, out_hbm, idx_v, rows_v, out_v, sem):
    wid = lax.axis_index("s") * SC_CORES + lax.axis_index("c")
    tpw = T_TOKENS // NW
    chunk = 32

    for c in range(tpw // chunk):
        tok0 = wid * tpw + c * chunk
        pltpu.sync_copy(pos_hbm.at[pl.ds(tok0 * TOP_K, chunk * TOP_K)], idx_v)
        pltpu.async_copy(ys_hbm.at[idx_v], rows_v, sem).wait()

        def tok_body(i, carry):
            for j in range(HIDDEN // LANES):
                a = rows_v[2 * i, pl.ds(j * LANES, LANES)]
                b = rows_v[2 * i + 1, pl.ds(j * LANES, LANES)]
                out_v[i, pl.ds(j * LANES, LANES)] = a + b
            return carry

        lax.fori_loop(0, chunk, tok_body, 0)
        pltpu.sync_copy(out_v, out_hbm.at[pl.ds(tok0, chunk)])


def kernel(hidden_states, gate_w, w1s, w2s, w3s):
    B, S, H = hidden_states.shape
    x = hidden_states.reshape(-1, H)

    topi, topw = pl.pallas_call(
        _router_body,
        grid=(T_TOKENS // TB,),
        in_specs=[
            pl.BlockSpec((TB, HIDDEN), lambda t: (t, 0)),
            pl.BlockSpec((HIDDEN, NUM_EXPERTS), lambda t: (0, 0)),
        ],
        out_specs=[
            pl.BlockSpec((TB, TOP_K), lambda t: (t, 0)),
            pl.BlockSpec((TB, TOP_K), lambda t: (t, 0)),
        ],
        out_shape=[
            jax.ShapeDtypeStruct((T_TOKENS, TOP_K), jnp.int32),
            jax.ShapeDtypeStruct((T_TOKENS, TOP_K), jnp.float32),
        ],
    )(x, gate_w)

    ef = topi.reshape(-1)
    wf = topw.reshape(-1)
    onehot = (ef[:, None] == jnp.arange(NUM_EXPERTS)[None, :]).astype(jnp.int32)
    csum = jnp.cumsum(onehot, axis=0)
    counts = csum[-1]
    pos_in_e = jnp.sum(onehot * csum, axis=1) - 1
    nb = (counts + BM - 1) // BM
    bstart = jnp.concatenate([jnp.zeros((1,), nb.dtype), jnp.cumsum(nb)[:-1]])
    pad_start = (bstart * BM).astype(jnp.int32)
    pos = jnp.sum(onehot * pad_start[None, :], axis=1) + pos_in_e
    pos_sw = jnp.transpose(pos.reshape(NW, TPW, TOP_K), (0, 2, 1))
    row_w = jnp.zeros((R,), jnp.float32).at[pos].set(wf, unique_indices=True)
    nb_total = bstart[-1] + nb[-1]
    gids = jnp.arange(G, dtype=jnp.int32)
    be = (jnp.sum(gids[:, None] >= bstart[None, :], axis=1) - 1).astype(jnp.int32)
    block_expert = jnp.where(gids < nb_total, be, NUM_EXPERTS)

    sc_distribute = pl.kernel(
        _sc_distribute_body,
        out_type=jax.ShapeDtypeStruct((R, HIDDEN), jnp.float32),
        mesh=plsc.VectorSubcoreMesh(core_axis_name="c", subcore_axis_name="s",
                               num_cores=SC_CORES, num_subcores=SC_SUBCORES),
        scratch_types=[
            pltpu.VMEM((TPW, HIDDEN), jnp.float32),
            pltpu.VMEM((TOP_K, TPW), jnp.int32),
            pltpu.SemaphoreType.DMA,
            pltpu.SemaphoreType.DMA,
        ],
    )
    xs = sc_distribute(x, pos_sw)

    grid_spec = pltpu.PrefetchScalarGridSpec(
        num_scalar_prefetch=1,
        grid=(G,),
        in_specs=[
            pl.BlockSpec((BM, HIDDEN), lambda g, be: (g, 0)),
            pl.BlockSpec((1, HIDDEN, INTER),
                         lambda g, be: (jnp.minimum(be[g], NUM_EXPERTS - 1), 0, 0)),
            pl.BlockSpec((1, HIDDEN, INTER),
                         lambda g, be: (jnp.minimum(be[g], NUM_EXPERTS - 1), 0, 0)),
            pl.BlockSpec((1, INTER, HIDDEN),
                         lambda g, be: (jnp.minimum(be[g], NUM_EXPERTS - 1), 0, 0)),
            pl.BlockSpec((BM, 1), lambda g, be: (g, 0)),
        ],
        out_specs=pl.BlockSpec((BM, HIDDEN), lambda g, be: (g, 0)),
    )
    ys = pl.pallas_call(
        _grouped_body,
        grid_spec=grid_spec,
        out_shape=jax.ShapeDtypeStruct((R, HIDDEN), jnp.float32),
        compiler_params=pltpu.CompilerParams(
            dimension_semantics=("arbitrary",),
        ),
    )(block_expert, xs, w1s, w3s, w2s, row_w.reshape(R, 1))

    sc_combine = pl.kernel(
        _sc_combine_body,
        out_type=jax.ShapeDtypeStruct((T_TOKENS, HIDDEN), jnp.float32),
        mesh=plsc.VectorSubcoreMesh(core_axis_name="c", subcore_axis_name="s",
                               num_cores=SC_CORES, num_subcores=SC_SUBCORES),
        scratch_types=[
            pltpu.VMEM((64,), jnp.int32),
            pltpu.VMEM((64, HIDDEN), jnp.float32),
            pltpu.VMEM((32, HIDDEN), jnp.float32),
            pltpu.SemaphoreType.DMA,
        ],
    )
    out = sc_combine(ys, pos.astype(jnp.int32))

    return out.reshape(B, S, H)


# speedup vs baseline: 1.3968x; 1.0592x over previous
import functools

import jax
import jax.numpy as jnp
from jax import lax
from jax.experimental import pallas as pl
from jax.experimental.pallas import tpu as pltpu
from jax.experimental.pallas import tpu_sc as plsc

NUM_EXPERTS = 8
TOP_K = 2
HIDDEN = 768
INTER = 2048
T_TOKENS = 2048
N_PAIRS = T_TOKENS * TOP_K

TB = 256
BM = 128
G = N_PAIRS // BM + NUM_EXPERTS
R = G * BM

SC_CORES = 2
SC_SUBCORES = 16
NW = SC_CORES * SC_SUBCORES
LANES = 16


def _router_body(x_ref, gw_ref, w_ref, pos_ref, be_ref):
    x = x_ref[...]
    logits = jnp.dot(x, gw_ref[...], preferred_element_type=jnp.float32)
    m = jnp.max(logits, axis=1, keepdims=True)
    ex = jnp.exp(logits - m)
    p = ex / jnp.sum(ex, axis=1, keepdims=True)
    iota = jax.lax.broadcasted_iota(jnp.int32, (p.shape[0], NUM_EXPERTS), 1)
    m1 = jnp.max(p, axis=1, keepdims=True)
    i1 = jnp.min(jnp.where(p == m1, iota, NUM_EXPERTS), axis=1, keepdims=True)
    is1 = iota == i1
    p2 = jnp.where(is1, -1.0, p)
    m2 = jnp.max(p2, axis=1, keepdims=True)
    i2 = jnp.min(jnp.where(p2 == m2, iota, NUM_EXPERTS), axis=1, keepdims=True)
    is2 = iota == i2
    s = m1 + m2
    w_ref[...] = jnp.concatenate([m1 / s, m2 / s], axis=1)

    oh = is1.astype(jnp.int32) + is2.astype(jnp.int32)
    cs = oh
    d = 1
    while d < T_TOKENS:
        shifted = jnp.concatenate(
            [jnp.zeros((d, NUM_EXPERTS), jnp.int32), cs[:T_TOKENS - d]], axis=0)
        cs = cs + shifted
        d *= 2
    before = cs - oh
    counts = cs[T_TOKENS - 1:T_TOKENS, :]
    nb = (counts + BM - 1) // BM
    bs = nb
    d = 1
    while d < NUM_EXPERTS:
        shifted = jnp.concatenate(
            [jnp.zeros((1, d), jnp.int32), bs[:, :NUM_EXPERTS - d]], axis=1)
        bs = bs + shifted
        d *= 2
    bstart = bs - nb
    pad_start = bstart * BM
    base = before + pad_start
    ps1 = jnp.sum(jnp.where(is1, base, 0), axis=1, keepdims=True)
    ps2 = jnp.sum(jnp.where(is2, base, 0), axis=1, keepdims=True)
    pos_ref[...] = jnp.concatenate([ps1, ps2], axis=1)
    nb_total = jnp.sum(nb, axis=1, keepdims=True)
    gi = jax.lax.broadcasted_iota(jnp.int32, (G, NUM_EXPERTS), 0)
    be = jnp.sum((gi >= bstart).astype(jnp.int32), axis=1, keepdims=True) - 1
    be_ref[...] = jnp.where(gi[:, :1] < nb_total, be, NUM_EXPERTS)


def _grouped_body(be_ref, xs_ref, w1_ref, w3_ref, w2_ref, rw_ref, ys_ref):
    g = pl.program_id(0)

    @pl.when(be_ref[g] < NUM_EXPERTS)
    def _():
        xb = xs_ref[...]
        h1 = jnp.dot(xb, w1_ref[0], preferred_element_type=jnp.float32)
        h3 = jnp.dot(xb, w3_ref[0], preferred_element_type=jnp.float32)
        hh = h1 * jax.nn.sigmoid(h1) * h3
        y = jnp.dot(hh, w2_ref[0], preferred_element_type=jnp.float32)
        ys_ref[...] = y * rw_ref[...]


TPW = T_TOKENS // NW


def _sc_distribute_body(x_hbm, pos_hbm, xs_hbm, xrows_v, idx_v, sem0, sem1):
    wid = lax.axis_index("s") * SC_CORES + lax.axis_index("c")
    pltpu.sync_copy(x_hbm.at[pl.ds(wid * TPW, TPW)], xrows_v)
    pltpu.sync_copy(pos_hbm.at[wid], idx_v)
    c0 = pltpu.async_copy(xrows_v, xs_hbm.at[idx_v.at[0]], sem0)
    c1 = pltpu.async_copy(xrows_v, xs_hbm.at[idx_v.at[1]], sem1)
    c0.wait()
    c1.wait()


def _sc_combine_body(ys_hbm, pos_hbm, out_hbm, idx_v, rows_v, out_v, sem):
    wid = lax.axis_index("s") * SC_CORES + lax.axis_index("c")
    tpw = T_TOKENS // NW
    chunk = 32

    for c in range(tpw // chunk):
        tok0 = wid * tpw + c * chunk
        pltpu.sync_copy(pos_hbm.at[pl.ds(tok0 * TOP_K, chunk * TOP_K)], idx_v)
        pltpu.async_copy(ys_hbm.at[idx_v], rows_v, sem).wait()

        def tok_body(i, carry):
            for j in range(HIDDEN // LANES):
                a = rows_v[2 * i, pl.ds(j * LANES, LANES)]
                b = rows_v[2 * i + 1, pl.ds(j * LANES, LANES)]
                out_v[i, pl.ds(j * LANES, LANES)] = a + b
            return carry

        lax.fori_loop(0, chunk, tok_body, 0)
        pltpu.sync_copy(out_v, out_hbm.at[pl.ds(tok0, chunk)])


def kernel(hidden_states, gate_w, w1s, w2s, w3s):
    B, S, H = hidden_states.shape
    x = hidden_states.reshape(-1, H)

    topw, pos2, be2 = pl.pallas_call(
        _router_body,
        grid=(1,),
        in_specs=[
            pl.BlockSpec((T_TOKENS, HIDDEN), lambda t: (0, 0)),
            pl.BlockSpec((HIDDEN, NUM_EXPERTS), lambda t: (0, 0)),
        ],
        out_specs=[
            pl.BlockSpec((T_TOKENS, TOP_K), lambda t: (0, 0)),
            pl.BlockSpec((T_TOKENS, TOP_K), lambda t: (0, 0)),
            pl.BlockSpec((G, 1), lambda t: (0, 0)),
        ],
        out_shape=[
            jax.ShapeDtypeStruct((T_TOKENS, TOP_K), jnp.float32),
            jax.ShapeDtypeStruct((T_TOKENS, TOP_K), jnp.int32),
            jax.ShapeDtypeStruct((G, 1), jnp.int32),
        ],
    )(x, gate_w)

    pos = pos2.reshape(-1)
    wf = topw.reshape(-1)
    pos_sw = jnp.transpose(pos2.reshape(NW, TPW, TOP_K), (0, 2, 1))
    row_w = jnp.zeros((R,), jnp.float32).at[pos].set(wf, unique_indices=True)
    block_expert = be2.reshape(-1)

    sc_distribute = pl.kernel(
        _sc_distribute_body,
        out_type=jax.ShapeDtypeStruct((R, HIDDEN), jnp.float32),
        mesh=plsc.VectorSubcoreMesh(core_axis_name="c", subcore_axis_name="s",
                               num_cores=SC_CORES, num_subcores=SC_SUBCORES),
        scratch_types=[
            pltpu.VMEM((TPW, HIDDEN), jnp.float32),
            pltpu.VMEM((TOP_K, TPW), jnp.int32),
            pltpu.SemaphoreType.DMA,
            pltpu.SemaphoreType.DMA,
        ],
    )
    xs = sc_distribute(x, pos_sw)

    grid_spec = pltpu.PrefetchScalarGridSpec(
        num_scalar_prefetch=1,
        grid=(G,),
        in_specs=[
            pl.BlockSpec((BM, HIDDEN), lambda g, be: (g, 0)),
            pl.BlockSpec((1, HIDDEN, INTER),
                         lambda g, be: (jnp.minimum(be[g], NUM_EXPERTS - 1), 0, 0)),
            pl.BlockSpec((1, HIDDEN, INTER),
                         lambda g, be: (jnp.minimum(be[g], NUM_EXPERTS - 1), 0, 0)),
            pl.BlockSpec((1, INTER, HIDDEN),
                         lambda g, be: (jnp.minimum(be[g], NUM_EXPERTS - 1), 0, 0)),
            pl.BlockSpec((BM, 1), lambda g, be: (g, 0)),
        ],
        out_specs=pl.BlockSpec((BM, HIDDEN), lambda g, be: (g, 0)),
    )
    ys = pl.pallas_call(
        _grouped_body,
        grid_spec=grid_spec,
        out_shape=jax.ShapeDtypeStruct((R, HIDDEN), jnp.float32),
        compiler_params=pltpu.CompilerParams(
            dimension_semantics=("arbitrary",),
        ),
    )(block_expert, xs, w1s, w3s, w2s, row_w.reshape(R, 1))

    sc_combine = pl.kernel(
        _sc_combine_body,
        out_type=jax.ShapeDtypeStruct((T_TOKENS, HIDDEN), jnp.float32),
        mesh=plsc.VectorSubcoreMesh(core_axis_name="c", subcore_axis_name="s",
                               num_cores=SC_CORES, num_subcores=SC_SUBCORES),
        scratch_types=[
            pltpu.VMEM((64,), jnp.int32),
            pltpu.VMEM((64, HIDDEN), jnp.float32),
            pltpu.VMEM((32, HIDDEN), jnp.float32),
            pltpu.SemaphoreType.DMA,
        ],
    )
    out = sc_combine(ys, pos.astype(jnp.int32))

    return out.reshape(B, S, H)


# speedup vs baseline: 1.4111x; 1.0102x over previous
import functools

import jax
import jax.numpy as jnp
from jax import lax
from jax.experimental import pallas as pl
from jax.experimental.pallas import tpu as pltpu
from jax.experimental.pallas import tpu_sc as plsc

NUM_EXPERTS = 8
TOP_K = 2
HIDDEN = 768
INTER = 2048
T_TOKENS = 2048
N_PAIRS = T_TOKENS * TOP_K

TB = 256
BM = 128
G = N_PAIRS // BM + NUM_EXPERTS
R = G * BM

SC_CORES = 2
SC_SUBCORES = 16
NW = SC_CORES * SC_SUBCORES
LANES = 16


def _router_body(x_ref, gw_ref, w_ref, pos_ref, be_ref):
    x = x_ref[...]
    logits = jnp.dot(x, gw_ref[...], preferred_element_type=jnp.float32)
    m = jnp.max(logits, axis=1, keepdims=True)
    ex = jnp.exp(logits - m)
    p = ex / jnp.sum(ex, axis=1, keepdims=True)
    iota = jax.lax.broadcasted_iota(jnp.int32, (p.shape[0], NUM_EXPERTS), 1)
    m1 = jnp.max(p, axis=1, keepdims=True)
    i1 = jnp.min(jnp.where(p == m1, iota, NUM_EXPERTS), axis=1, keepdims=True)
    is1 = iota == i1
    p2 = jnp.where(is1, -1.0, p)
    m2 = jnp.max(p2, axis=1, keepdims=True)
    i2 = jnp.min(jnp.where(p2 == m2, iota, NUM_EXPERTS), axis=1, keepdims=True)
    is2 = iota == i2
    s = m1 + m2
    w_ref[...] = jnp.concatenate([m1 / s, m2 / s], axis=1)

    oh = is1.astype(jnp.int32) + is2.astype(jnp.int32)
    cs = oh
    d = 1
    while d < T_TOKENS:
        shifted = jnp.concatenate(
            [jnp.zeros((d, NUM_EXPERTS), jnp.int32), cs[:T_TOKENS - d]], axis=0)
        cs = cs + shifted
        d *= 2
    before = cs - oh
    counts = cs[T_TOKENS - 1:T_TOKENS, :]
    nb = (counts + BM - 1) // BM
    bs = nb
    d = 1
    while d < NUM_EXPERTS:
        shifted = jnp.concatenate(
            [jnp.zeros((1, d), jnp.int32), bs[:, :NUM_EXPERTS - d]], axis=1)
        bs = bs + shifted
        d *= 2
    bstart = bs - nb
    pad_start = bstart * BM
    base = before + pad_start
    ps1 = jnp.sum(jnp.where(is1, base, 0), axis=1, keepdims=True)
    ps2 = jnp.sum(jnp.where(is2, base, 0), axis=1, keepdims=True)
    pos_ref[...] = jnp.concatenate([ps1, ps2], axis=1)
    nb_total = jnp.sum(nb, axis=1, keepdims=True)
    gi = jax.lax.broadcasted_iota(jnp.int32, (G, NUM_EXPERTS), 0)
    be = jnp.sum((gi >= bstart).astype(jnp.int32), axis=1, keepdims=True) - 1
    be_ref[...] = jnp.where(gi[:, :1] < nb_total, be, NUM_EXPERTS)


def _grouped_body(be_ref, xs_ref, w1_ref, w3_ref, w2_ref, rw_ref, ys_ref):
    g = pl.program_id(0)

    @pl.when(be_ref[g] < NUM_EXPERTS)
    def _():
        xb = xs_ref[...]
        h1 = jnp.dot(xb, w1_ref[0], preferred_element_type=jnp.float32)
        h3 = jnp.dot(xb, w3_ref[0], preferred_element_type=jnp.float32)
        hh = h1 * jax.nn.sigmoid(h1) * h3
        y = jnp.dot(hh, w2_ref[0], preferred_element_type=jnp.float32)
        ys_ref[...] = y * rw_ref[...]


TPW = T_TOKENS // NW


def _sc_distribute_body(x_hbm, pos_hbm, xs_hbm, xrows_v, idx_v, sem0, sem1):
    wid = lax.axis_index("s") * SC_CORES + lax.axis_index("c")
    pltpu.sync_copy(x_hbm.at[pl.ds(wid * TPW, TPW)], xrows_v)
    pltpu.sync_copy(pos_hbm.at[wid], idx_v)
    c0 = pltpu.async_copy(xrows_v, xs_hbm.at[idx_v.at[0]], sem0)
    c1 = pltpu.async_copy(xrows_v, xs_hbm.at[idx_v.at[1]], sem1)
    c0.wait()
    c1.wait()


CCH = 4
CTOK = T_TOKENS // NW // CCH


def _sc_combine_body(ys_hbm, pos_hbm, out_hbm, idx_v, r0, r1, r2, r3,
                     ob0, ob1, g0, g1, g2, g3, wsem):
    wid = lax.axis_index("s") * SC_CORES + lax.axis_index("c")
    pltpu.sync_copy(pos_hbm.at[pl.ds(wid * CCH, CCH)], idx_v)
    bufs = (r0, r1, r2, r3)
    gsems = (g0, g1, g2, g3)
    outb = (ob0, ob1)
    gath = [pltpu.async_copy(ys_hbm.at[idx_v.at[c]], bufs[c], gsems[c])
            for c in range(CCH)]
    writes = [None, None]
    for c in range(CCH):
        rows_v = bufs[c]
        out_v = outb[c % 2]
        if writes[c % 2] is not None:
            writes[c % 2].wait()
        gath[c].wait()

        def tok_body(i, carry):
            for j in range(HIDDEN // LANES):
                a = rows_v[2 * i, pl.ds(j * LANES, LANES)]
                b = rows_v[2 * i + 1, pl.ds(j * LANES, LANES)]
                out_v[i, pl.ds(j * LANES, LANES)] = a + b
            return carry

        lax.fori_loop(0, CTOK, tok_body, 0)
        writes[c % 2] = pltpu.async_copy(
            out_v, out_hbm.at[pl.ds(wid * (CCH * CTOK) + c * CTOK, CTOK)], wsem)
    writes[0].wait()
    writes[1].wait()


def kernel(hidden_states, gate_w, w1s, w2s, w3s):
    B, S, H = hidden_states.shape
    x = hidden_states.reshape(-1, H)

    topw, pos2, be2 = pl.pallas_call(
        _router_body,
        grid=(1,),
        in_specs=[
            pl.BlockSpec((T_TOKENS, HIDDEN), lambda t: (0, 0)),
            pl.BlockSpec((HIDDEN, NUM_EXPERTS), lambda t: (0, 0)),
        ],
        out_specs=[
            pl.BlockSpec((T_TOKENS, TOP_K), lambda t: (0, 0)),
            pl.BlockSpec((T_TOKENS, TOP_K), lambda t: (0, 0)),
            pl.BlockSpec((G, 1), lambda t: (0, 0)),
        ],
        out_shape=[
            jax.ShapeDtypeStruct((T_TOKENS, TOP_K), jnp.float32),
            jax.ShapeDtypeStruct((T_TOKENS, TOP_K), jnp.int32),
            jax.ShapeDtypeStruct((G, 1), jnp.int32),
        ],
    )(x, gate_w)

    pos = pos2.reshape(-1)
    wf = topw.reshape(-1)
    pos_sw = jnp.transpose(pos2.reshape(NW, TPW, TOP_K), (0, 2, 1))
    row_w = jnp.zeros((R,), jnp.float32).at[pos].set(wf, unique_indices=True)
    block_expert = be2.reshape(-1)

    sc_distribute = pl.kernel(
        _sc_distribute_body,
        out_type=jax.ShapeDtypeStruct((R, HIDDEN), jnp.float32),
        mesh=plsc.VectorSubcoreMesh(core_axis_name="c", subcore_axis_name="s",
                               num_cores=SC_CORES, num_subcores=SC_SUBCORES),
        scratch_types=[
            pltpu.VMEM((TPW, HIDDEN), jnp.float32),
            pltpu.VMEM((TOP_K, TPW), jnp.int32),
            pltpu.SemaphoreType.DMA,
            pltpu.SemaphoreType.DMA,
        ],
    )
    xs = sc_distribute(x, pos_sw)

    grid_spec = pltpu.PrefetchScalarGridSpec(
        num_scalar_prefetch=1,
        grid=(G,),
        in_specs=[
            pl.BlockSpec((BM, HIDDEN), lambda g, be: (g, 0)),
            pl.BlockSpec((1, HIDDEN, INTER),
                         lambda g, be: (jnp.minimum(be[g], NUM_EXPERTS - 1), 0, 0)),
            pl.BlockSpec((1, HIDDEN, INTER),
                         lambda g, be: (jnp.minimum(be[g], NUM_EXPERTS - 1), 0, 0)),
            pl.BlockSpec((1, INTER, HIDDEN),
                         lambda g, be: (jnp.minimum(be[g], NUM_EXPERTS - 1), 0, 0)),
            pl.BlockSpec((BM, 1), lambda g, be: (g, 0)),
        ],
        out_specs=pl.BlockSpec((BM, HIDDEN), lambda g, be: (g, 0)),
    )
    ys = pl.pallas_call(
        _grouped_body,
        grid_spec=grid_spec,
        out_shape=jax.ShapeDtypeStruct((R, HIDDEN), jnp.float32),
        compiler_params=pltpu.CompilerParams(
            dimension_semantics=("arbitrary",),
        ),
    )(block_expert, xs, w1s, w3s, w2s, row_w.reshape(R, 1))

    sc_combine = pl.kernel(
        _sc_combine_body,
        out_type=jax.ShapeDtypeStruct((T_TOKENS, HIDDEN), jnp.float32),
        mesh=plsc.VectorSubcoreMesh(core_axis_name="c", subcore_axis_name="s",
                               num_cores=SC_CORES, num_subcores=SC_SUBCORES),
        scratch_types=[
            pltpu.VMEM((CCH, TOP_K * CTOK), jnp.int32),
            pltpu.VMEM((TOP_K * CTOK, HIDDEN), jnp.float32),
            pltpu.VMEM((TOP_K * CTOK, HIDDEN), jnp.float32),
            pltpu.VMEM((TOP_K * CTOK, HIDDEN), jnp.float32),
            pltpu.VMEM((TOP_K * CTOK, HIDDEN), jnp.float32),
            pltpu.VMEM((CTOK, HIDDEN), jnp.float32),
            pltpu.VMEM((CTOK, HIDDEN), jnp.float32),
            pltpu.SemaphoreType.DMA,
            pltpu.SemaphoreType.DMA,
            pltpu.SemaphoreType.DMA,
            pltpu.SemaphoreType.DMA,
            pltpu.SemaphoreType.DMA,
        ],
    )
    out = sc_combine(ys, pos.reshape(NW * CCH, TOP_K * CTOK))

    return out.reshape(B, S, H)
